# Initial kernel scaffold; baseline (speedup 1.0000x reference)
import jax
import jax.numpy as jnp
from jax.experimental import pallas as pl


def kernel(embeddings, W1, b1, W2, b2, W3, b3, ln_scale, ln_bias, W4, b4, W5, b5):
    raise NotImplementedError("write your pallas kernel here")


# speedup vs baseline: 8.3382x; 8.3382x over previous
import jax
import jax.numpy as jnp
from jax.experimental import pallas as pl
from jax.experimental.pallas import tpu as pltpu

_B, _S, _D, _T, _K = 4, 2048, 1024, 512, 32
_BLK = 256


def _prep_kernel(emb_ref, w1_ref, b1_ref, norm_ref, topo32_ref, topo16_ref):
    e = emb_ref[0]
    n = e / (jnp.sqrt(jnp.sum(e * e, axis=1, keepdims=True)) + 1e-8)
    norm_ref[0] = n.astype(jnp.bfloat16)
    t = jax.lax.dot_general(
        e.astype(jnp.bfloat16), w1_ref[...], (((1,), (0,)), ((), ())),
        preferred_element_type=jnp.float32) + b1_ref[...]
    topo32_ref[0] = t
    topo16_ref[0] = t.astype(jnp.bfloat16)


def _main_kernel(nrm_ref, nrow_ref, topo16_ref, trow_ref,
                 w2_ref, b2_ref, w3_ref, b3_ref, lns_ref, lnb_ref,
                 w4_ref, b4_ref, w5_ref, b5_ref,
                 nd_ref, ni_ref, pf_ref,
                 dist_ref):
    i = pl.program_id(1)
    rows = nrow_ref[0]
    sim = jax.lax.dot_general(
        rows, nrm_ref[0], (((1,), (1,)), ((), ())),
        preferred_element_type=jnp.float32)
    dist = 1.0 - sim
    row_ids = i * _BLK + jax.lax.broadcasted_iota(jnp.int32, (_BLK, _S), 0)
    col_ids = jax.lax.broadcasted_iota(jnp.int32, (_BLK, _S), 1)
    dist_ref[...] = jnp.where(col_ids == row_ids, 1e9, dist)
    k_iota = jax.lax.broadcasted_iota(jnp.int32, (_BLK, _K), 1)

    def body(t, carry):
        nd, ni = carry
        d = dist_ref[...]
        m = jnp.min(d, axis=1, keepdims=True)
        cand = jnp.where(d == m, col_ids, _S)
        idx = jnp.min(cand, axis=1, keepdims=True)
        sel = cand == idx
        dist_ref[...] = jnp.where(sel, jnp.float32(jnp.inf), d)
        nd = jnp.where(k_iota == t, m, nd)
        ni = jnp.where(k_iota == t, idx, ni)
        return nd, ni

    nd, ni = jax.lax.fori_loop(
        0, _K, body,
        (jnp.zeros((_BLK, _K), jnp.float32),
         jnp.zeros((_BLK, _K), jnp.int32)))

    d0 = nd[:, 0:1]
    e = jnp.exp(d0 - nd)
    z = jnp.sum(e, axis=1, keepdims=True)
    w = e / z
    a = jnp.zeros((_BLK, _S), jnp.float32)
    for t in range(_K):
        a = jnp.where(col_ids == ni[:, t:t + 1], w[:, t:t + 1], a)
    abf = a.astype(jnp.bfloat16)
    wn = jax.lax.dot_general(
        abf, topo16_ref[0], (((1,), (0,)), ((), ())),
        preferred_element_type=jnp.float32)
    comb = trow_ref[0] + wn

    x = jax.lax.dot_general(
        comb.astype(jnp.bfloat16), w2_ref[...], (((1,), (0,)), ((), ())),
        preferred_element_type=jnp.float32) + b2_ref[...]
    x = jnp.maximum(x, 0.0)
    x = jax.lax.dot_general(
        x.astype(jnp.bfloat16), w3_ref[...], (((1,), (0,)), ((), ())),
        preferred_element_type=jnp.float32) + b3_ref[...]
    mu = jnp.mean(x, axis=1, keepdims=True)
    xc = x - mu
    var = jnp.mean(xc * xc, axis=1, keepdims=True)
    tf = xc / jnp.sqrt(var + 1e-6) * lns_ref[...] + lnb_ref[...]
    x = jax.lax.dot_general(
        tf.astype(jnp.bfloat16), w4_ref[...], (((1,), (0,)), ((), ())),
        preferred_element_type=jnp.float32) + b4_ref[...]
    x = jnp.maximum(x, 0.0)
    pf = jax.lax.dot_general(
        x.astype(jnp.bfloat16), w5_ref[...], (((1,), (0,)), ((), ())),
        preferred_element_type=jnp.float32) + b5_ref[...]

    nd_ref[0] = nd
    ni_ref[0] = ni
    pf_ref[0] = pf


def kernel(embeddings, W1, b1, W2, b2, W3, b3, ln_scale, ln_bias, W4, b4, W5, b5):
    f32, bf16 = jnp.float32, jnp.bfloat16
    norm16, topo32, topo16 = pl.pallas_call(
        _prep_kernel,
        grid=(_B,),
        in_specs=[
            pl.BlockSpec((1, _S, _D), lambda b: (b, 0, 0)),
            pl.BlockSpec((_D, _T), lambda b: (0, 0)),
            pl.BlockSpec((1, _T), lambda b: (0, 0)),
        ],
        out_specs=[
            pl.BlockSpec((1, _S, _D), lambda b: (b, 0, 0)),
            pl.BlockSpec((1, _S, _T), lambda b: (b, 0, 0)),
            pl.BlockSpec((1, _S, _T), lambda b: (b, 0, 0)),
        ],
        out_shape=[
            jax.ShapeDtypeStruct((_B, _S, _D), bf16),
            jax.ShapeDtypeStruct((_B, _S, _T), f32),
            jax.ShapeDtypeStruct((_B, _S, _T), bf16),
        ],
    )(embeddings, W1.astype(bf16), b1.reshape(1, _T))

    full = lambda shape: pl.BlockSpec(shape, lambda b, i: tuple(0 for _ in shape))
    nd, ni, pf = pl.pallas_call(
        _main_kernel,
        grid=(_B, _S // _BLK),
        in_specs=[
            pl.BlockSpec((1, _S, _D), lambda b, i: (b, 0, 0)),
            pl.BlockSpec((1, _BLK, _D), lambda b, i: (b, i, 0)),
            pl.BlockSpec((1, _S, _T), lambda b, i: (b, 0, 0)),
            pl.BlockSpec((1, _BLK, _T), lambda b, i: (b, i, 0)),
            full((_T, 2 * _T)), full((1, 2 * _T)),
            full((2 * _T, _T)), full((1, _T)),
            full((1, _T)), full((1, _T)),
            full((_T, _T)), full((1, _T)),
            full((_T, _T)), full((1, _T)),
        ],
        out_specs=[
            pl.BlockSpec((1, _BLK, _K), lambda b, i: (b, i, 0)),
            pl.BlockSpec((1, _BLK, _K), lambda b, i: (b, i, 0)),
            pl.BlockSpec((1, _BLK, _T), lambda b, i: (b, i, 0)),
        ],
        out_shape=[
            jax.ShapeDtypeStruct((_B, _S, _K), f32),
            jax.ShapeDtypeStruct((_B, _S, _K), jnp.int32),
            jax.ShapeDtypeStruct((_B, _S, _T), f32),
        ],
        scratch_shapes=[
            pltpu.VMEM((_BLK, _S), f32),
        ],
    )(norm16, norm16, topo16, topo32,
      W2.astype(bf16), b2.reshape(1, 2 * _T),
      W3.astype(bf16), b3.reshape(1, _T),
      ln_scale.reshape(1, _T), ln_bias.reshape(1, _T),
      W4.astype(bf16), b4.reshape(1, _T),
      W5.astype(bf16), b5.reshape(1, _T))
    return pf, nd, ni


# speedup vs baseline: 10.5405x; 1.2641x over previous
import functools

import jax
import jax.numpy as jnp
from jax import lax
from jax.experimental import pallas as pl
from jax.experimental.pallas import tpu as pltpu
from jax.experimental.pallas import tpu_sc as plsc

_B, _S, _D, _T, _K = 4, 2048, 1024, 512, 32
_BLK = 256
_DBLK = 512
_R = _B * _S
_NW = 32
_RPW = _R // _NW
_GRP = 16
_NG = _RPW // _GRP
_CW = 32
_NCH = _S // _CW
_BIGF = 3.0e38


def _prep_kernel(emb_ref, w1_ref, b1_ref, norm_ref, topo32_ref, topo16_ref):
    e = emb_ref[0]
    n = e / (jnp.sqrt(jnp.sum(e * e, axis=1, keepdims=True)) + 1e-8)
    norm_ref[0] = n.astype(jnp.bfloat16)
    t = jax.lax.dot_general(
        e.astype(jnp.bfloat16), w1_ref[...], (((1,), (0,)), ((), ())),
        preferred_element_type=jnp.float32) + b1_ref[...]
    topo32_ref[0] = t
    topo16_ref[0] = t.astype(jnp.bfloat16)


def _dist_kernel(nrm_ref, nrow_ref, dist_ref):
    i = pl.program_id(1)
    sim = jax.lax.dot_general(
        nrow_ref[0], nrm_ref[0], (((1,), (1,)), ((), ())),
        preferred_element_type=jnp.float32)
    dist = 1.0 - sim
    row_ids = i * _DBLK + jax.lax.broadcasted_iota(jnp.int32, (_DBLK, _S), 0)
    col_ids = jax.lax.broadcasted_iota(jnp.int32, (_DBLK, _S), 1)
    dist_ref[0] = jnp.where(col_ids == row_ids, 1e9, dist)


def _sc_topk_kernel(dist_hbm, nd_hbm, ni_hbm, buf, cm, ndv, niv):
    wid = lax.axis_index("s") * 2 + lax.axis_index("c")
    lane = lax.iota(jnp.int32, 16)
    rowoff = lane * _S
    zeros_i = jnp.zeros((16,), jnp.int32)
    bigv = jnp.full((16,), _BIGF, jnp.float32)

    def group_body(g, carry_g):
        base = wid * _RPW + g * _GRP
        pltpu.sync_copy(dist_hbm.at[pl.ds(base * _S, _GRP * _S)], buf)

        def cm_body(c, carry):
            def o_body(o, acc):
                a0 = acc
                idx = rowoff + (c * _CW + o * 4)
                for u in range(4):
                    v = plsc.load_gather(buf, [idx + u])
                    a0 = jnp.minimum(a0, v)
                return a0
            acc = lax.fori_loop(0, _CW // 4, o_body, bigv)
            cm[pl.ds(c * _GRP, _GRP)] = acc
            return carry
        lax.fori_loop(0, _NCH, cm_body, 0)

        def t_body(t, carry_t):
            def am_body(c, carry):
                bv, bc = carry
                for u in range(4):
                    v = cm[pl.ds((c * 4 + u) * _GRP, _GRP)]
                    lt = v < bv
                    bv = jnp.where(lt, v, bv)
                    bc = jnp.where(lt, (c * 4 + u) + zeros_i, bc)
                return bv, bc
            m, cidx = lax.fori_loop(0, _NCH // 4, am_body, (bigv, zeros_i))
            colbase = cidx * _CW

            def rs_body(o, carry):
                b1, bcol, b2 = carry
                for u in range(4):
                    col = colbase + (o * 4 + u)
                    v = plsc.load_gather(buf, [rowoff + col])
                    lt = v < b1
                    b2 = jnp.where(lt, b1, jnp.minimum(b2, v))
                    b1 = jnp.where(lt, v, b1)
                    bcol = jnp.where(lt, col, bcol)
                return b1, bcol, b2
            m1, col, nmin = lax.fori_loop(
                0, _CW // 4, rs_body, (bigv, zeros_i, bigv))

            plsc.store_scatter(buf, [rowoff + col], bigv)
            plsc.store_scatter(cm, [cidx * _GRP + lane], nmin)
            tv = t + zeros_i
            plsc.store_scatter(ndv, [lane * _K + tv], m1)
            plsc.store_scatter(niv, [lane * _K + tv], col)
            return carry_t
        lax.fori_loop(0, _K, t_body, 0)

        pltpu.sync_copy(ndv, nd_hbm.at[pl.ds(base * _K, _GRP * _K)])
        pltpu.sync_copy(niv, ni_hbm.at[pl.ds(base * _K, _GRP * _K)])
        return carry_g
    lax.fori_loop(0, _NG, group_body, 0)


def _tail_kernel(nd_ref, ni_ref, topo16_ref, trow_ref,
                 w2_ref, b2_ref, w3_ref, b3_ref, lns_ref, lnb_ref,
                 w4_ref, b4_ref, w5_ref, b5_ref, pf_ref):
    nd = nd_ref[0]
    ni = ni_ref[0]
    col_ids = jax.lax.broadcasted_iota(jnp.int32, (_BLK, _S), 1)
    d0 = nd[:, 0:1]
    e = jnp.exp(d0 - nd)
    z = jnp.sum(e, axis=1, keepdims=True)
    w = e / z
    a = jnp.zeros((_BLK, _S), jnp.float32)
    for t in range(_K):
        a = jnp.where(col_ids == ni[:, t:t + 1], w[:, t:t + 1], a)
    abf = a.astype(jnp.bfloat16)
    wn = jax.lax.dot_general(
        abf, topo16_ref[0], (((1,), (0,)), ((), ())),
        preferred_element_type=jnp.float32)
    comb = trow_ref[0] + wn

    x = jax.lax.dot_general(
        comb.astype(jnp.bfloat16), w2_ref[...], (((1,), (0,)), ((), ())),
        preferred_element_type=jnp.float32) + b2_ref[...]
    x = jnp.maximum(x, 0.0)
    x = jax.lax.dot_general(
        x.astype(jnp.bfloat16), w3_ref[...], (((1,), (0,)), ((), ())),
        preferred_element_type=jnp.float32) + b3_ref[...]
    mu = jnp.mean(x, axis=1, keepdims=True)
    xc = x - mu
    var = jnp.mean(xc * xc, axis=1, keepdims=True)
    tf = xc / jnp.sqrt(var + 1e-6) * lns_ref[...] + lnb_ref[...]
    x = jax.lax.dot_general(
        tf.astype(jnp.bfloat16), w4_ref[...], (((1,), (0,)), ((), ())),
        preferred_element_type=jnp.float32) + b4_ref[...]
    x = jnp.maximum(x, 0.0)
    pf = jax.lax.dot_general(
        x.astype(jnp.bfloat16), w5_ref[...], (((1,), (0,)), ((), ())),
        preferred_element_type=jnp.float32) + b5_ref[...]
    pf_ref[0] = pf


_sc_topk = functools.partial(
    pl.kernel,
    out_type=[
        jax.ShapeDtypeStruct((_R * _K,), jnp.float32),
        jax.ShapeDtypeStruct((_R * _K,), jnp.int32),
    ],
    mesh=plsc.VectorSubcoreMesh(core_axis_name="c", subcore_axis_name="s"),
    compiler_params=pltpu.CompilerParams(needs_layout_passes=False),
    scratch_types=[
        pltpu.VMEM((_GRP * _S,), jnp.float32),
        pltpu.VMEM((_NCH * _GRP,), jnp.float32),
        pltpu.VMEM((_GRP * _K,), jnp.float32),
        pltpu.VMEM((_GRP * _K,), jnp.int32),
    ],
)(_sc_topk_kernel)


def kernel(embeddings, W1, b1, W2, b2, W3, b3, ln_scale, ln_bias, W4, b4, W5, b5):
    f32, bf16 = jnp.float32, jnp.bfloat16
    norm16, topo32, topo16 = pl.pallas_call(
        _prep_kernel,
        grid=(_B,),
        in_specs=[
            pl.BlockSpec((1, _S, _D), lambda b: (b, 0, 0)),
            pl.BlockSpec((_D, _T), lambda b: (0, 0)),
            pl.BlockSpec((1, _T), lambda b: (0, 0)),
        ],
        out_specs=[
            pl.BlockSpec((1, _S, _D), lambda b: (b, 0, 0)),
            pl.BlockSpec((1, _S, _T), lambda b: (b, 0, 0)),
            pl.BlockSpec((1, _S, _T), lambda b: (b, 0, 0)),
        ],
        out_shape=[
            jax.ShapeDtypeStruct((_B, _S, _D), bf16),
            jax.ShapeDtypeStruct((_B, _S, _T), f32),
            jax.ShapeDtypeStruct((_B, _S, _T), bf16),
        ],
    )(embeddings, W1.astype(bf16), b1.reshape(1, _T))

    dist = pl.pallas_call(
        _dist_kernel,
        grid=(_B, _S // _DBLK),
        in_specs=[
            pl.BlockSpec((1, _S, _D), lambda b, i: (b, 0, 0)),
            pl.BlockSpec((1, _DBLK, _D), lambda b, i: (b, i, 0)),
        ],
        out_specs=pl.BlockSpec((1, _DBLK, _S), lambda b, i: (b, i, 0)),
        out_shape=jax.ShapeDtypeStruct((_B, _S, _S), f32),
    )(norm16, norm16)

    nd, ni = _sc_topk(dist.reshape(_R * _S))
    nd = nd.reshape(_B, _S, _K)
    ni = ni.reshape(_B, _S, _K)

    full = lambda shape: pl.BlockSpec(shape, lambda b, i: tuple(0 for _ in shape))
    pf = pl.pallas_call(
        _tail_kernel,
        grid=(_B, _S // _BLK),
        in_specs=[
            pl.BlockSpec((1, _BLK, _K), lambda b, i: (b, i, 0)),
            pl.BlockSpec((1, _BLK, _K), lambda b, i: (b, i, 0)),
            pl.BlockSpec((1, _S, _T), lambda b, i: (b, 0, 0)),
            pl.BlockSpec((1, _BLK, _T), lambda b, i: (b, i, 0)),
            full((_T, 2 * _T)), full((1, 2 * _T)),
            full((2 * _T, _T)), full((1, _T)),
            full((1, _T)), full((1, _T)),
            full((_T, _T)), full((1, _T)),
            full((_T, _T)), full((1, _T)),
        ],
        out_specs=pl.BlockSpec((1, _BLK, _T), lambda b, i: (b, i, 0)),
        out_shape=jax.ShapeDtypeStruct((_B, _S, _T), f32),
    )(nd, ni, topo16, topo32,
      W2.astype(bf16), b2.reshape(1, 2 * _T),
      W3.astype(bf16), b3.reshape(1, _T),
      ln_scale.reshape(1, _T), ln_bias.reshape(1, _T),
      W4.astype(bf16), b4.reshape(1, _T),
      W5.astype(bf16), b5.reshape(1, _T))
    return pf, nd, ni


# speedup vs baseline: 11.2223x; 1.0647x over previous
import functools

import jax
import jax.numpy as jnp
from jax import lax
from jax.experimental import pallas as pl
from jax.experimental.pallas import tpu as pltpu
from jax.experimental.pallas import tpu_sc as plsc

_B, _S, _D, _T, _K = 4, 2048, 1024, 512, 32
_BLK = 256
_DBLK = 512
_R = _B * _S
_NW = 32
_RPW = _R // _NW
_GRP = 16
_NG = _RPW // _GRP
_CW = 32
_NCH = _S // _CW
_BIGF = 3.0e38


def _prep_kernel(emb_ref, w1_ref, b1_ref, norm_ref, topo32_ref, topo16_ref):
    e = emb_ref[0]
    n = e / (jnp.sqrt(jnp.sum(e * e, axis=1, keepdims=True)) + 1e-8)
    norm_ref[0] = n.astype(jnp.bfloat16)
    t = jax.lax.dot_general(
        e.astype(jnp.bfloat16), w1_ref[...], (((1,), (0,)), ((), ())),
        preferred_element_type=jnp.float32) + b1_ref[...]
    topo32_ref[0] = t
    topo16_ref[0] = t.astype(jnp.bfloat16)


def _dist_kernel(nrm_ref, nrow_ref, dist_ref):
    i = pl.program_id(1)
    sim = jax.lax.dot_general(
        nrow_ref[0], nrm_ref[0], (((1,), (1,)), ((), ())),
        preferred_element_type=jnp.float32)
    dist = 1.0 - sim
    row_ids = i * _DBLK + jax.lax.broadcasted_iota(jnp.int32, (_DBLK, _S), 0)
    col_ids = jax.lax.broadcasted_iota(jnp.int32, (_DBLK, _S), 1)
    dist_ref[0] = jnp.where(col_ids == row_ids, 1e9, dist)


def _sc_topk_kernel(dist_hbm, nd_hbm, ni_hbm, buf, cm, ndv, niv, sem):
    wid = lax.axis_index("s") * 2 + lax.axis_index("c")
    batch = wid // (_S // _RPW)
    rowbase = (wid % (_S // _RPW)) * _RPW
    lane = lax.iota(jnp.int32, 16)
    rowoff = lane * _S
    zeros_i = jnp.zeros((16,), jnp.int32)
    bigv = jnp.full((16,), _BIGF, jnp.float32)

    def group_body(g, carry_g):
        base = wid * _RPW + g * _GRP
        copies = [
            pltpu.async_copy(
                dist_hbm.at[batch, rowbase + g * _GRP + l],
                buf.at[pl.ds(l * _S, _S)], sem)
            for l in range(_GRP)
        ]
        for cp in copies:
            cp.wait()

        def cm_body(c, carry):
            def o_body(o, acc):
                a0 = acc
                idx = rowoff + (c * _CW + o * 4)
                for u in range(4):
                    v = plsc.load_gather(buf, [idx + u])
                    a0 = jnp.minimum(a0, v)
                return a0
            acc = lax.fori_loop(0, _CW // 4, o_body, bigv)
            cm[pl.ds(c * _GRP, _GRP)] = acc
            return carry
        lax.fori_loop(0, _NCH, cm_body, 0)

        def t_body(t, carry_t):
            def am_body(c, carry):
                bv, bc = carry
                for u in range(4):
                    v = cm[pl.ds((c * 4 + u) * _GRP, _GRP)]
                    lt = v < bv
                    bv = jnp.where(lt, v, bv)
                    bc = jnp.where(lt, (c * 4 + u) + zeros_i, bc)
                return bv, bc
            m, cidx = lax.fori_loop(0, _NCH // 4, am_body, (bigv, zeros_i))
            colbase = cidx * _CW

            def rs_body(o, carry):
                b1, bcol, b2 = carry
                for u in range(4):
                    col = colbase + (o * 4 + u)
                    v = plsc.load_gather(buf, [rowoff + col])
                    lt = v < b1
                    b2 = jnp.where(lt, b1, jnp.minimum(b2, v))
                    b1 = jnp.where(lt, v, b1)
                    bcol = jnp.where(lt, col, bcol)
                return b1, bcol, b2
            m1, col, nmin = lax.fori_loop(
                0, _CW // 4, rs_body, (bigv, zeros_i, bigv))

            plsc.store_scatter(buf, [rowoff + col], bigv)
            plsc.store_scatter(cm, [cidx * _GRP + lane], nmin)
            tv = t + zeros_i
            plsc.store_scatter(ndv, [lane * _K + tv], m1)
            plsc.store_scatter(niv, [lane * _K + tv], col)
            return carry_t
        lax.fori_loop(0, _K, t_body, 0)

        pltpu.sync_copy(ndv, nd_hbm.at[pl.ds(base * _K, _GRP * _K)])
        pltpu.sync_copy(niv, ni_hbm.at[pl.ds(base * _K, _GRP * _K)])
        return carry_g
    lax.fori_loop(0, _NG, group_body, 0)


def _tail_kernel(nd_ref, ni_ref, topo16_ref, trow_ref,
                 w2_ref, b2_ref, w3_ref, b3_ref, lns_ref, lnb_ref,
                 w4_ref, b4_ref, w5_ref, b5_ref, pf_ref):
    nd = nd_ref[0]
    ni = ni_ref[0]
    col_ids = jax.lax.broadcasted_iota(jnp.int32, (_BLK, _S), 1)
    d0 = nd[:, 0:1]
    e = jnp.exp(d0 - nd)
    z = jnp.sum(e, axis=1, keepdims=True)
    w = e / z
    a = jnp.zeros((_BLK, _S), jnp.float32)
    for t in range(_K):
        a = jnp.where(col_ids == ni[:, t:t + 1], w[:, t:t + 1], a)
    abf = a.astype(jnp.bfloat16)
    wn = jax.lax.dot_general(
        abf, topo16_ref[0], (((1,), (0,)), ((), ())),
        preferred_element_type=jnp.float32)
    comb = trow_ref[0] + wn

    x = jax.lax.dot_general(
        comb.astype(jnp.bfloat16), w2_ref[...], (((1,), (0,)), ((), ())),
        preferred_element_type=jnp.float32) + b2_ref[...]
    x = jnp.maximum(x, 0.0)
    x = jax.lax.dot_general(
        x.astype(jnp.bfloat16), w3_ref[...], (((1,), (0,)), ((), ())),
        preferred_element_type=jnp.float32) + b3_ref[...]
    mu = jnp.mean(x, axis=1, keepdims=True)
    xc = x - mu
    var = jnp.mean(xc * xc, axis=1, keepdims=True)
    tf = xc / jnp.sqrt(var + 1e-6) * lns_ref[...] + lnb_ref[...]
    x = jax.lax.dot_general(
        tf.astype(jnp.bfloat16), w4_ref[...], (((1,), (0,)), ((), ())),
        preferred_element_type=jnp.float32) + b4_ref[...]
    x = jnp.maximum(x, 0.0)
    pf = jax.lax.dot_general(
        x.astype(jnp.bfloat16), w5_ref[...], (((1,), (0,)), ((), ())),
        preferred_element_type=jnp.float32) + b5_ref[...]
    pf_ref[0] = pf


_sc_topk = functools.partial(
    pl.kernel,
    out_type=[
        jax.ShapeDtypeStruct((_R * _K,), jnp.float32),
        jax.ShapeDtypeStruct((_R * _K,), jnp.int32),
    ],
    mesh=plsc.VectorSubcoreMesh(core_axis_name="c", subcore_axis_name="s"),
    compiler_params=pltpu.CompilerParams(needs_layout_passes=False),
    scratch_types=[
        pltpu.VMEM((_GRP * _S,), jnp.float32),
        pltpu.VMEM((_NCH * _GRP,), jnp.float32),
        pltpu.VMEM((_GRP * _K,), jnp.float32),
        pltpu.VMEM((_GRP * _K,), jnp.int32),
        pltpu.SemaphoreType.DMA,
    ],
)(_sc_topk_kernel)


def kernel(embeddings, W1, b1, W2, b2, W3, b3, ln_scale, ln_bias, W4, b4, W5, b5):
    f32, bf16 = jnp.float32, jnp.bfloat16
    norm16, topo32, topo16 = pl.pallas_call(
        _prep_kernel,
        grid=(_B,),
        in_specs=[
            pl.BlockSpec((1, _S, _D), lambda b: (b, 0, 0)),
            pl.BlockSpec((_D, _T), lambda b: (0, 0)),
            pl.BlockSpec((1, _T), lambda b: (0, 0)),
        ],
        out_specs=[
            pl.BlockSpec((1, _S, _D), lambda b: (b, 0, 0)),
            pl.BlockSpec((1, _S, _T), lambda b: (b, 0, 0)),
            pl.BlockSpec((1, _S, _T), lambda b: (b, 0, 0)),
        ],
        out_shape=[
            jax.ShapeDtypeStruct((_B, _S, _D), bf16),
            jax.ShapeDtypeStruct((_B, _S, _T), f32),
            jax.ShapeDtypeStruct((_B, _S, _T), bf16),
        ],
    )(embeddings, W1.astype(bf16), b1.reshape(1, _T))

    dist = pl.pallas_call(
        _dist_kernel,
        grid=(_B, _S // _DBLK),
        in_specs=[
            pl.BlockSpec((1, _S, _D), lambda b, i: (b, 0, 0)),
            pl.BlockSpec((1, _DBLK, _D), lambda b, i: (b, i, 0)),
        ],
        out_specs=pl.BlockSpec((1, _DBLK, _S), lambda b, i: (b, i, 0)),
        out_shape=jax.ShapeDtypeStruct((_B, _S, _S), f32),
    )(norm16, norm16)

    nd, ni = _sc_topk(dist)
    nd = nd.reshape(_B, _S, _K)
    ni = ni.reshape(_B, _S, _K)

    full = lambda shape: pl.BlockSpec(shape, lambda b, i: tuple(0 for _ in shape))
    pf = pl.pallas_call(
        _tail_kernel,
        grid=(_B, _S // _BLK),
        in_specs=[
            pl.BlockSpec((1, _BLK, _K), lambda b, i: (b, i, 0)),
            pl.BlockSpec((1, _BLK, _K), lambda b, i: (b, i, 0)),
            pl.BlockSpec((1, _S, _T), lambda b, i: (b, 0, 0)),
            pl.BlockSpec((1, _BLK, _T), lambda b, i: (b, i, 0)),
            full((_T, 2 * _T)), full((1, 2 * _T)),
            full((2 * _T, _T)), full((1, _T)),
            full((1, _T)), full((1, _T)),
            full((_T, _T)), full((1, _T)),
            full((_T, _T)), full((1, _T)),
        ],
        out_specs=pl.BlockSpec((1, _BLK, _T), lambda b, i: (b, i, 0)),
        out_shape=jax.ShapeDtypeStruct((_B, _S, _T), f32),
    )(nd, ni, topo16, topo32,
      W2.astype(bf16), b2.reshape(1, 2 * _T),
      W3.astype(bf16), b3.reshape(1, _T),
      ln_scale.reshape(1, _T), ln_bias.reshape(1, _T),
      W4.astype(bf16), b4.reshape(1, _T),
      W5.astype(bf16), b5.reshape(1, _T))
    return pf, nd, ni


# speedup vs baseline: 11.5025x; 1.0250x over previous
import functools

import jax
import jax.numpy as jnp
from jax import lax
from jax.experimental import pallas as pl
from jax.experimental.pallas import tpu as pltpu
from jax.experimental.pallas import tpu_sc as plsc

_B, _S, _D, _T, _K = 4, 2048, 1024, 512, 32
_BLK = 256
_DBLK = 512
_R = _B * _S
_NW = 32
_RPW = _R // _NW
_GRP = 16
_NG = _RPW // _GRP
_CW = 32
_NCH = _S // _CW
_BIGF = 3.0e38


def _prep_kernel(emb_ref, w1_ref, b1_ref, norm_ref, topo32_ref, topo16_ref):
    e = emb_ref[0]
    n = e / (jnp.sqrt(jnp.sum(e * e, axis=1, keepdims=True)) + 1e-8)
    norm_ref[0] = n.astype(jnp.bfloat16)
    t = jax.lax.dot_general(
        e.astype(jnp.bfloat16), w1_ref[...], (((1,), (0,)), ((), ())),
        preferred_element_type=jnp.float32) + b1_ref[...]
    topo32_ref[0] = t
    topo16_ref[0] = t.astype(jnp.bfloat16)


def _dist_kernel(nrm_ref, nrow_ref, dist_ref):
    i = pl.program_id(1)
    sim = jax.lax.dot_general(
        nrow_ref[0], nrm_ref[0], (((1,), (1,)), ((), ())),
        preferred_element_type=jnp.float32)
    dist = 1.0 - sim
    row_ids = i * _DBLK + jax.lax.broadcasted_iota(jnp.int32, (_DBLK, _S), 0)
    col_ids = jax.lax.broadcasted_iota(jnp.int32, (_DBLK, _S), 1)
    dist_ref[0] = jnp.where(col_ids == row_ids, 1e9, dist)


def _sc_topk_kernel(dist_hbm, nd_hbm, ni_hbm, buf, cm, ndv, niv, sem):
    wid = lax.axis_index("s") * 2 + lax.axis_index("c")
    batch = wid // (_S // _RPW)
    rowbase = (wid % (_S // _RPW)) * _RPW
    lane = lax.iota(jnp.int32, 16)
    rowoff = lane * _S
    zeros_i = jnp.zeros((16,), jnp.int32)
    bigv = jnp.full((16,), _BIGF, jnp.float32)

    def group_body(g, carry_g):
        base = wid * _RPW + g * _GRP
        copies = [
            pltpu.async_copy(
                dist_hbm.at[batch, rowbase + g * _GRP + l],
                buf.at[pl.ds(l * _S, _S)], sem)
            for l in range(_GRP)
        ]
        for cp in copies:
            cp.wait()

        def cm_body(c, carry):
            idx = rowoff + c * _CW
            acc = bigv
            for u in range(_CW):
                v = plsc.load_gather(buf, [idx + u])
                acc = jnp.minimum(acc, v)
            cm[pl.ds(c * _GRP, _GRP)] = acc
            return carry
        lax.fori_loop(0, _NCH, cm_body, 0)

        def t_body(t, carry_t):
            bv, bc = bigv, zeros_i
            for c in range(_NCH):
                v = cm[pl.ds(c * _GRP, _GRP)]
                lt = v < bv
                bv = jnp.where(lt, v, bv)
                bc = jnp.where(lt, c + zeros_i, bc)
            m, cidx = bv, bc
            colbase = cidx * _CW

            b1, bcol, b2 = bigv, zeros_i, bigv
            for u in range(_CW):
                col = colbase + u
                v = plsc.load_gather(buf, [rowoff + col])
                lt = v < b1
                b2 = jnp.where(lt, b1, jnp.minimum(b2, v))
                b1 = jnp.where(lt, v, b1)
                bcol = jnp.where(lt, col, bcol)
            m1, col, nmin = b1, bcol, b2

            plsc.store_scatter(buf, [rowoff + col], bigv)
            plsc.store_scatter(cm, [cidx * _GRP + lane], nmin)
            tv = t + zeros_i
            plsc.store_scatter(ndv, [lane * _K + tv], m1)
            plsc.store_scatter(niv, [lane * _K + tv], col)
            return carry_t
        lax.fori_loop(0, _K, t_body, 0)

        pltpu.sync_copy(ndv, nd_hbm.at[pl.ds(base * _K, _GRP * _K)])
        pltpu.sync_copy(niv, ni_hbm.at[pl.ds(base * _K, _GRP * _K)])
        return carry_g
    lax.fori_loop(0, _NG, group_body, 0)


def _tail_kernel(nd_ref, ni_ref, topo16_ref, trow_ref,
                 w2_ref, b2_ref, w3_ref, b3_ref, lns_ref, lnb_ref,
                 w4_ref, b4_ref, w5_ref, b5_ref, pf_ref):
    nd = nd_ref[0]
    ni = ni_ref[0]
    col_ids = jax.lax.broadcasted_iota(jnp.int32, (_BLK, _S), 1)
    d0 = nd[:, 0:1]
    e = jnp.exp(d0 - nd)
    z = jnp.sum(e, axis=1, keepdims=True)
    w = e / z
    a = jnp.zeros((_BLK, _S), jnp.float32)
    for t in range(_K):
        a = jnp.where(col_ids == ni[:, t:t + 1], w[:, t:t + 1], a)
    abf = a.astype(jnp.bfloat16)
    wn = jax.lax.dot_general(
        abf, topo16_ref[0], (((1,), (0,)), ((), ())),
        preferred_element_type=jnp.float32)
    comb = trow_ref[0] + wn

    x = jax.lax.dot_general(
        comb.astype(jnp.bfloat16), w2_ref[...], (((1,), (0,)), ((), ())),
        preferred_element_type=jnp.float32) + b2_ref[...]
    x = jnp.maximum(x, 0.0)
    x = jax.lax.dot_general(
        x.astype(jnp.bfloat16), w3_ref[...], (((1,), (0,)), ((), ())),
        preferred_element_type=jnp.float32) + b3_ref[...]
    mu = jnp.mean(x, axis=1, keepdims=True)
    xc = x - mu
    var = jnp.mean(xc * xc, axis=1, keepdims=True)
    tf = xc / jnp.sqrt(var + 1e-6) * lns_ref[...] + lnb_ref[...]
    x = jax.lax.dot_general(
        tf.astype(jnp.bfloat16), w4_ref[...], (((1,), (0,)), ((), ())),
        preferred_element_type=jnp.float32) + b4_ref[...]
    x = jnp.maximum(x, 0.0)
    pf = jax.lax.dot_general(
        x.astype(jnp.bfloat16), w5_ref[...], (((1,), (0,)), ((), ())),
        preferred_element_type=jnp.float32) + b5_ref[...]
    pf_ref[0] = pf


_sc_topk = functools.partial(
    pl.kernel,
    out_type=[
        jax.ShapeDtypeStruct((_R * _K,), jnp.float32),
        jax.ShapeDtypeStruct((_R * _K,), jnp.int32),
    ],
    mesh=plsc.VectorSubcoreMesh(core_axis_name="c", subcore_axis_name="s"),
    compiler_params=pltpu.CompilerParams(needs_layout_passes=False),
    scratch_types=[
        pltpu.VMEM((_GRP * _S,), jnp.float32),
        pltpu.VMEM((_NCH * _GRP,), jnp.float32),
        pltpu.VMEM((_GRP * _K,), jnp.float32),
        pltpu.VMEM((_GRP * _K,), jnp.int32),
        pltpu.SemaphoreType.DMA,
    ],
)(_sc_topk_kernel)


def kernel(embeddings, W1, b1, W2, b2, W3, b3, ln_scale, ln_bias, W4, b4, W5, b5):
    f32, bf16 = jnp.float32, jnp.bfloat16
    norm16, topo32, topo16 = pl.pallas_call(
        _prep_kernel,
        grid=(_B,),
        in_specs=[
            pl.BlockSpec((1, _S, _D), lambda b: (b, 0, 0)),
            pl.BlockSpec((_D, _T), lambda b: (0, 0)),
            pl.BlockSpec((1, _T), lambda b: (0, 0)),
        ],
        out_specs=[
            pl.BlockSpec((1, _S, _D), lambda b: (b, 0, 0)),
            pl.BlockSpec((1, _S, _T), lambda b: (b, 0, 0)),
            pl.BlockSpec((1, _S, _T), lambda b: (b, 0, 0)),
        ],
        out_shape=[
            jax.ShapeDtypeStruct((_B, _S, _D), bf16),
            jax.ShapeDtypeStruct((_B, _S, _T), f32),
            jax.ShapeDtypeStruct((_B, _S, _T), bf16),
        ],
    )(embeddings, W1.astype(bf16), b1.reshape(1, _T))

    dist = pl.pallas_call(
        _dist_kernel,
        grid=(_B, _S // _DBLK),
        in_specs=[
            pl.BlockSpec((1, _S, _D), lambda b, i: (b, 0, 0)),
            pl.BlockSpec((1, _DBLK, _D), lambda b, i: (b, i, 0)),
        ],
        out_specs=pl.BlockSpec((1, _DBLK, _S), lambda b, i: (b, i, 0)),
        out_shape=jax.ShapeDtypeStruct((_B, _S, _S), f32),
    )(norm16, norm16)

    nd, ni = _sc_topk(dist)
    nd = nd.reshape(_B, _S, _K)
    ni = ni.reshape(_B, _S, _K)

    full = lambda shape: pl.BlockSpec(shape, lambda b, i: tuple(0 for _ in shape))
    pf = pl.pallas_call(
        _tail_kernel,
        grid=(_B, _S // _BLK),
        in_specs=[
            pl.BlockSpec((1, _BLK, _K), lambda b, i: (b, i, 0)),
            pl.BlockSpec((1, _BLK, _K), lambda b, i: (b, i, 0)),
            pl.BlockSpec((1, _S, _T), lambda b, i: (b, 0, 0)),
            pl.BlockSpec((1, _BLK, _T), lambda b, i: (b, i, 0)),
            full((_T, 2 * _T)), full((1, 2 * _T)),
            full((2 * _T, _T)), full((1, _T)),
            full((1, _T)), full((1, _T)),
            full((_T, _T)), full((1, _T)),
            full((_T, _T)), full((1, _T)),
        ],
        out_specs=pl.BlockSpec((1, _BLK, _T), lambda b, i: (b, i, 0)),
        out_shape=jax.ShapeDtypeStruct((_B, _S, _T), f32),
    )(nd, ni, topo16, topo32,
      W2.astype(bf16), b2.reshape(1, 2 * _T),
      W3.astype(bf16), b3.reshape(1, _T),
      ln_scale.reshape(1, _T), ln_bias.reshape(1, _T),
      W4.astype(bf16), b4.reshape(1, _T),
      W5.astype(bf16), b5.reshape(1, _T))
    return pf, nd, ni


# speedup vs baseline: 11.5364x; 1.0029x over previous
import functools

import jax
import jax.numpy as jnp
from jax import lax
from jax.experimental import pallas as pl
from jax.experimental.pallas import tpu as pltpu
from jax.experimental.pallas import tpu_sc as plsc

_B, _S, _D, _T, _K = 4, 2048, 1024, 512, 32
_BLK = 256
_DBLK = 512
_R = _B * _S
_NW = 32
_RPW = _R // _NW
_GRP = 16
_NG = _RPW // _GRP
_CW = 32
_NCH = _S // _CW
_BIGF = 3.0e38


def _prep_kernel(emb_ref, w1_ref, b1_ref, topo32_ref, topo16_ref):
    e = emb_ref[0]
    t = jax.lax.dot_general(
        e.astype(jnp.bfloat16), w1_ref[...], (((1,), (0,)), ((), ())),
        preferred_element_type=jnp.float32) + b1_ref[...]
    topo32_ref[0] = t
    topo16_ref[0] = t.astype(jnp.bfloat16)


def _dist_kernel(nrm_ref, nrow_ref, dist_ref):
    i = pl.program_id(1)
    sim = jax.lax.dot_general(
        nrow_ref[0], nrm_ref[0], (((1,), (1,)), ((), ())),
        preferred_element_type=jnp.float32)
    dist = 1.0 - sim
    row_ids = i * _DBLK + jax.lax.broadcasted_iota(jnp.int32, (_DBLK, _S), 0)
    col_ids = jax.lax.broadcasted_iota(jnp.int32, (_DBLK, _S), 1)
    dist_ref[0] = jnp.where(col_ids == row_ids, 1e9, dist)


def _sc_topk_kernel(dist_hbm, nd_hbm, ni_hbm, buf, cm, ndv, niv, sem):
    wid = lax.axis_index("s") * 2 + lax.axis_index("c")
    batch = wid // (_S // _RPW)
    rowbase = (wid % (_S // _RPW)) * _RPW
    lane = lax.iota(jnp.int32, 16)
    rowoff = lane * _S
    zeros_i = jnp.zeros((16,), jnp.int32)
    bigv = jnp.full((16,), _BIGF, jnp.float32)

    def group_body(g, carry_g):
        base = wid * _RPW + g * _GRP
        copies = [
            pltpu.async_copy(
                dist_hbm.at[batch, rowbase + g * _GRP + l],
                buf.at[pl.ds(l * _S, _S)], sem)
            for l in range(_GRP)
        ]
        for cp in copies:
            cp.wait()

        def cm_body(c, carry):
            idx = rowoff + c * _CW
            acc = bigv
            for u in range(_CW):
                v = plsc.load_gather(buf, [idx + u])
                acc = jnp.minimum(acc, v)
            cm[pl.ds(c * _GRP, _GRP)] = acc
            return carry
        lax.fori_loop(0, _NCH, cm_body, 0)

        def t_body(t, carry_t):
            bv, bc = bigv, zeros_i
            for c in range(_NCH):
                v = cm[pl.ds(c * _GRP, _GRP)]
                lt = v < bv
                bv = jnp.where(lt, v, bv)
                bc = jnp.where(lt, c + zeros_i, bc)
            m, cidx = bv, bc
            colbase = cidx * _CW

            b1, bcol, b2 = bigv, zeros_i, bigv
            for u in range(_CW):
                col = colbase + u
                v = plsc.load_gather(buf, [rowoff + col])
                lt = v < b1
                b2 = jnp.where(lt, b1, jnp.minimum(b2, v))
                b1 = jnp.where(lt, v, b1)
                bcol = jnp.where(lt, col, bcol)
            m1, col, nmin = b1, bcol, b2

            plsc.store_scatter(buf, [rowoff + col], bigv)
            plsc.store_scatter(cm, [cidx * _GRP + lane], nmin)
            tv = t + zeros_i
            plsc.store_scatter(ndv, [lane * _K + tv], m1)
            plsc.store_scatter(niv, [lane * _K + tv], col)
            return carry_t
        lax.fori_loop(0, _K, t_body, 0)

        pltpu.sync_copy(ndv, nd_hbm.at[pl.ds(base * _K, _GRP * _K)])
        pltpu.sync_copy(niv, ni_hbm.at[pl.ds(base * _K, _GRP * _K)])
        return carry_g
    lax.fori_loop(0, _NG, group_body, 0)


def _tail_kernel(nd_ref, ni_ref, topo16_ref, trow_ref,
                 w2_ref, b2_ref, w3_ref, b3_ref, lns_ref, lnb_ref,
                 w4_ref, b4_ref, w5_ref, b5_ref, pf_ref):
    nd = nd_ref[0]
    ni = ni_ref[0]
    col_ids = jax.lax.broadcasted_iota(jnp.int32, (_BLK, _S), 1)
    d0 = nd[:, 0:1]
    e = jnp.exp(d0 - nd)
    z = jnp.sum(e, axis=1, keepdims=True)
    w = e / z
    a = jnp.zeros((_BLK, _S), jnp.float32)
    for t in range(_K):
        a = jnp.where(col_ids == ni[:, t:t + 1], w[:, t:t + 1], a)
    abf = a.astype(jnp.bfloat16)
    wn = jax.lax.dot_general(
        abf, topo16_ref[0], (((1,), (0,)), ((), ())),
        preferred_element_type=jnp.float32)
    comb = trow_ref[0] + wn

    x = jax.lax.dot_general(
        comb.astype(jnp.bfloat16), w2_ref[...], (((1,), (0,)), ((), ())),
        preferred_element_type=jnp.float32) + b2_ref[...]
    x = jnp.maximum(x, 0.0)
    x = jax.lax.dot_general(
        x.astype(jnp.bfloat16), w3_ref[...], (((1,), (0,)), ((), ())),
        preferred_element_type=jnp.float32) + b3_ref[...]
    mu = jnp.mean(x, axis=1, keepdims=True)
    xc = x - mu
    var = jnp.mean(xc * xc, axis=1, keepdims=True)
    tf = xc / jnp.sqrt(var + 1e-6) * lns_ref[...] + lnb_ref[...]
    x = jax.lax.dot_general(
        tf.astype(jnp.bfloat16), w4_ref[...], (((1,), (0,)), ((), ())),
        preferred_element_type=jnp.float32) + b4_ref[...]
    x = jnp.maximum(x, 0.0)
    pf = jax.lax.dot_general(
        x.astype(jnp.bfloat16), w5_ref[...], (((1,), (0,)), ((), ())),
        preferred_element_type=jnp.float32) + b5_ref[...]
    pf_ref[0] = pf


_sc_topk = functools.partial(
    pl.kernel,
    out_type=[
        jax.ShapeDtypeStruct((_R * _K,), jnp.float32),
        jax.ShapeDtypeStruct((_R * _K,), jnp.int32),
    ],
    mesh=plsc.VectorSubcoreMesh(core_axis_name="c", subcore_axis_name="s"),
    compiler_params=pltpu.CompilerParams(needs_layout_passes=False),
    scratch_types=[
        pltpu.VMEM((_GRP * _S,), jnp.float32),
        pltpu.VMEM((_NCH * _GRP,), jnp.float32),
        pltpu.VMEM((_GRP * _K,), jnp.float32),
        pltpu.VMEM((_GRP * _K,), jnp.int32),
        pltpu.SemaphoreType.DMA,
    ],
)(_sc_topk_kernel)


def kernel(embeddings, W1, b1, W2, b2, W3, b3, ln_scale, ln_bias, W4, b4, W5, b5):
    f32, bf16 = jnp.float32, jnp.bfloat16
    norm16 = (embeddings / (jnp.linalg.norm(
        embeddings, axis=-1, keepdims=True) + 1e-08)).astype(bf16)
    topo32, topo16 = pl.pallas_call(
        _prep_kernel,
        grid=(_B,),
        in_specs=[
            pl.BlockSpec((1, _S, _D), lambda b: (b, 0, 0)),
            pl.BlockSpec((_D, _T), lambda b: (0, 0)),
            pl.BlockSpec((1, _T), lambda b: (0, 0)),
        ],
        out_specs=[
            pl.BlockSpec((1, _S, _T), lambda b: (b, 0, 0)),
            pl.BlockSpec((1, _S, _T), lambda b: (b, 0, 0)),
        ],
        out_shape=[
            jax.ShapeDtypeStruct((_B, _S, _T), f32),
            jax.ShapeDtypeStruct((_B, _S, _T), bf16),
        ],
    )(embeddings, W1.astype(bf16), b1.reshape(1, _T))

    dist = pl.pallas_call(
        _dist_kernel,
        grid=(_B, _S // _DBLK),
        in_specs=[
            pl.BlockSpec((1, _S, _D), lambda b, i: (b, 0, 0)),
            pl.BlockSpec((1, _DBLK, _D), lambda b, i: (b, i, 0)),
        ],
        out_specs=pl.BlockSpec((1, _DBLK, _S), lambda b, i: (b, i, 0)),
        out_shape=jax.ShapeDtypeStruct((_B, _S, _S), f32),
    )(norm16, norm16)

    nd, ni = _sc_topk(dist)
    nd = nd.reshape(_B, _S, _K)
    ni = ni.reshape(_B, _S, _K)

    full = lambda shape: pl.BlockSpec(shape, lambda b, i: tuple(0 for _ in shape))
    pf = pl.pallas_call(
        _tail_kernel,
        grid=(_B, _S // _BLK),
        in_specs=[
            pl.BlockSpec((1, _BLK, _K), lambda b, i: (b, i, 0)),
            pl.BlockSpec((1, _BLK, _K), lambda b, i: (b, i, 0)),
            pl.BlockSpec((1, _S, _T), lambda b, i: (b, 0, 0)),
            pl.BlockSpec((1, _BLK, _T), lambda b, i: (b, i, 0)),
            full((_T, 2 * _T)), full((1, 2 * _T)),
            full((2 * _T, _T)), full((1, _T)),
            full((1, _T)), full((1, _T)),
            full((_T, _T)), full((1, _T)),
            full((_T, _T)), full((1, _T)),
        ],
        out_specs=pl.BlockSpec((1, _BLK, _T), lambda b, i: (b, i, 0)),
        out_shape=jax.ShapeDtypeStruct((_B, _S, _T), f32),
    )(nd, ni, topo16, topo32,
      W2.astype(bf16), b2.reshape(1, 2 * _T),
      W3.astype(bf16), b3.reshape(1, _T),
      ln_scale.reshape(1, _T), ln_bias.reshape(1, _T),
      W4.astype(bf16), b4.reshape(1, _T),
      W5.astype(bf16), b5.reshape(1, _T))
    return pf, nd, ni


# speedup vs baseline: 11.6439x; 1.0093x over previous
import functools

import jax
import jax.numpy as jnp
from jax import lax
from jax.experimental import pallas as pl
from jax.experimental.pallas import tpu as pltpu
from jax.experimental.pallas import tpu_sc as plsc

_B, _S, _D, _T, _K = 4, 2048, 1024, 512, 32
_BLK = 256
_DBLK = 512
_R = _B * _S
_NW = 32
_RPW = _R // _NW
_GRP = 16
_NG = _RPW // _GRP
_CW = 32
_NCH = _S // _CW
_BIGF = 3.0e38


def _prep_kernel(emb_ref, w1_ref, b1_ref, topo32_ref, topo16_ref):
    e = emb_ref[0]
    t = jax.lax.dot_general(
        e.astype(jnp.bfloat16), w1_ref[...], (((1,), (0,)), ((), ())),
        preferred_element_type=jnp.float32) + b1_ref[...]
    topo32_ref[0] = t
    topo16_ref[0] = t.astype(jnp.bfloat16)


def _dist_kernel(nrm_ref, nrow_ref, dist_ref):
    i = pl.program_id(1)
    sim = jax.lax.dot_general(
        nrow_ref[0], nrm_ref[0], (((1,), (1,)), ((), ())),
        preferred_element_type=jnp.float32)
    dist = 1.0 - sim
    row_ids = i * _DBLK + jax.lax.broadcasted_iota(jnp.int32, (_DBLK, _S), 0)
    col_ids = jax.lax.broadcasted_iota(jnp.int32, (_DBLK, _S), 1)
    dist_ref[0] = jnp.where(col_ids == row_ids, 1e9, dist)


def _sc_topk_kernel(dist_hbm, nd_hbm, ni_hbm, buf, cm, ndv, niv, sem):
    wid = lax.axis_index("s") * 2 + lax.axis_index("c")
    batch = wid // (_S // _RPW)
    rowbase = (wid % (_S // _RPW)) * _RPW
    lane = lax.iota(jnp.int32, 16)
    rowoff = lane * _S
    zeros_i = jnp.zeros((16,), jnp.int32)
    bigv = jnp.full((16,), _BIGF, jnp.float32)

    def group_body(g, carry_g):
        base = wid * _RPW + g * _GRP
        copies = [
            pltpu.async_copy(
                dist_hbm.at[batch, rowbase + g * _GRP + l],
                buf.at[pl.ds(l * _S, _S)], sem)
            for l in range(_GRP)
        ]
        for cp in copies:
            cp.wait()

        def cm_body(c, carry):
            idx = rowoff + c * _CW
            accs = [bigv] * 4
            for u in range(_CW):
                v = plsc.load_gather(buf, [idx + u])
                accs[u % 4] = jnp.minimum(accs[u % 4], v)
            acc = jnp.minimum(jnp.minimum(accs[0], accs[1]),
                              jnp.minimum(accs[2], accs[3]))
            cm[pl.ds(c * _GRP, _GRP)] = acc
            return carry
        lax.fori_loop(0, _NCH, cm_body, 0)

        def t_body(t, carry_t):
            bvs, bcs = [bigv] * 4, [zeros_i] * 4
            for c in range(_NCH):
                j = c % 4
                v = cm[pl.ds(c * _GRP, _GRP)]
                lt = v < bvs[j]
                bvs[j] = jnp.where(lt, v, bvs[j])
                bcs[j] = jnp.where(lt, c + zeros_i, bcs[j])
            m, cidx = bvs[0], bcs[0]
            for j in range(1, 4):
                better = (bvs[j] < m) | ((bvs[j] == m) & (bcs[j] < cidx))
                m = jnp.where(better, bvs[j], m)
                cidx = jnp.where(better, bcs[j], cidx)
            colbase = cidx * _CW

            st = [[bigv, zeros_i, bigv] for _ in range(4)]
            for u in range(_CW):
                j = u % 4
                col = colbase + u
                v = plsc.load_gather(buf, [rowoff + col])
                b1, bcol, b2 = st[j]
                lt = v < b1
                st[j][2] = jnp.where(lt, b1, jnp.minimum(b2, v))
                st[j][0] = jnp.where(lt, v, b1)
                st[j][1] = jnp.where(lt, col, bcol)
            m1, col, nmin = st[0]
            for j in range(1, 4):
                bvv, bc2, b2v = st[j]
                better = (bvv < m1) | ((bvv == m1) & (bc2 < col))
                nmin = jnp.where(better, jnp.minimum(b2v, m1),
                                 jnp.minimum(nmin, bvv))
                m1 = jnp.where(better, bvv, m1)
                col = jnp.where(better, bc2, col)

            plsc.store_scatter(buf, [rowoff + col], bigv)
            plsc.store_scatter(cm, [cidx * _GRP + lane], nmin)
            tv = t + zeros_i
            plsc.store_scatter(ndv, [lane * _K + tv], m1)
            plsc.store_scatter(niv, [lane * _K + tv], col)
            return carry_t
        lax.fori_loop(0, _K, t_body, 0)

        pltpu.sync_copy(ndv, nd_hbm.at[pl.ds(base * _K, _GRP * _K)])
        pltpu.sync_copy(niv, ni_hbm.at[pl.ds(base * _K, _GRP * _K)])
        return carry_g
    lax.fori_loop(0, _NG, group_body, 0)


def _tail_kernel(nd_ref, ni_ref, topo16_ref, trow_ref,
                 w2_ref, b2_ref, w3_ref, b3_ref, lns_ref, lnb_ref,
                 w4_ref, b4_ref, w5_ref, b5_ref, pf_ref):
    nd = nd_ref[0]
    ni = ni_ref[0]
    col_ids = jax.lax.broadcasted_iota(jnp.int32, (_BLK, _S), 1)
    d0 = nd[:, 0:1]
    e = jnp.exp(d0 - nd)
    z = jnp.sum(e, axis=1, keepdims=True)
    w = e / z
    a = jnp.zeros((_BLK, _S), jnp.float32)
    for t in range(_K):
        a = jnp.where(col_ids == ni[:, t:t + 1], w[:, t:t + 1], a)
    abf = a.astype(jnp.bfloat16)
    wn = jax.lax.dot_general(
        abf, topo16_ref[0], (((1,), (0,)), ((), ())),
        preferred_element_type=jnp.float32)
    comb = trow_ref[0] + wn

    x = jax.lax.dot_general(
        comb.astype(jnp.bfloat16), w2_ref[...], (((1,), (0,)), ((), ())),
        preferred_element_type=jnp.float32) + b2_ref[...]
    x = jnp.maximum(x, 0.0)
    x = jax.lax.dot_general(
        x.astype(jnp.bfloat16), w3_ref[...], (((1,), (0,)), ((), ())),
        preferred_element_type=jnp.float32) + b3_ref[...]
    mu = jnp.mean(x, axis=1, keepdims=True)
    xc = x - mu
    var = jnp.mean(xc * xc, axis=1, keepdims=True)
    tf = xc / jnp.sqrt(var + 1e-6) * lns_ref[...] + lnb_ref[...]
    x = jax.lax.dot_general(
        tf.astype(jnp.bfloat16), w4_ref[...], (((1,), (0,)), ((), ())),
        preferred_element_type=jnp.float32) + b4_ref[...]
    x = jnp.maximum(x, 0.0)
    pf = jax.lax.dot_general(
        x.astype(jnp.bfloat16), w5_ref[...], (((1,), (0,)), ((), ())),
        preferred_element_type=jnp.float32) + b5_ref[...]
    pf_ref[0] = pf


_sc_topk = functools.partial(
    pl.kernel,
    out_type=[
        jax.ShapeDtypeStruct((_R * _K,), jnp.float32),
        jax.ShapeDtypeStruct((_R * _K,), jnp.int32),
    ],
    mesh=plsc.VectorSubcoreMesh(core_axis_name="c", subcore_axis_name="s"),
    compiler_params=pltpu.CompilerParams(needs_layout_passes=False),
    scratch_types=[
        pltpu.VMEM((_GRP * _S,), jnp.float32),
        pltpu.VMEM((_NCH * _GRP,), jnp.float32),
        pltpu.VMEM((_GRP * _K,), jnp.float32),
        pltpu.VMEM((_GRP * _K,), jnp.int32),
        pltpu.SemaphoreType.DMA,
    ],
)(_sc_topk_kernel)


def kernel(embeddings, W1, b1, W2, b2, W3, b3, ln_scale, ln_bias, W4, b4, W5, b5):
    f32, bf16 = jnp.float32, jnp.bfloat16
    norm16 = (embeddings / (jnp.linalg.norm(
        embeddings, axis=-1, keepdims=True) + 1e-08)).astype(bf16)
    topo32, topo16 = pl.pallas_call(
        _prep_kernel,
        grid=(_B,),
        in_specs=[
            pl.BlockSpec((1, _S, _D), lambda b: (b, 0, 0)),
            pl.BlockSpec((_D, _T), lambda b: (0, 0)),
            pl.BlockSpec((1, _T), lambda b: (0, 0)),
        ],
        out_specs=[
            pl.BlockSpec((1, _S, _T), lambda b: (b, 0, 0)),
            pl.BlockSpec((1, _S, _T), lambda b: (b, 0, 0)),
        ],
        out_shape=[
            jax.ShapeDtypeStruct((_B, _S, _T), f32),
            jax.ShapeDtypeStruct((_B, _S, _T), bf16),
        ],
    )(embeddings, W1.astype(bf16), b1.reshape(1, _T))

    dist = pl.pallas_call(
        _dist_kernel,
        grid=(_B, _S // _DBLK),
        in_specs=[
            pl.BlockSpec((1, _S, _D), lambda b, i: (b, 0, 0)),
            pl.BlockSpec((1, _DBLK, _D), lambda b, i: (b, i, 0)),
        ],
        out_specs=pl.BlockSpec((1, _DBLK, _S), lambda b, i: (b, i, 0)),
        out_shape=jax.ShapeDtypeStruct((_B, _S, _S), f32),
    )(norm16, norm16)

    nd, ni = _sc_topk(dist)
    nd = nd.reshape(_B, _S, _K)
    ni = ni.reshape(_B, _S, _K)

    full = lambda shape: pl.BlockSpec(shape, lambda b, i: tuple(0 for _ in shape))
    pf = pl.pallas_call(
        _tail_kernel,
        grid=(_B, _S // _BLK),
        in_specs=[
            pl.BlockSpec((1, _BLK, _K), lambda b, i: (b, i, 0)),
            pl.BlockSpec((1, _BLK, _K), lambda b, i: (b, i, 0)),
            pl.BlockSpec((1, _S, _T), lambda b, i: (b, 0, 0)),
            pl.BlockSpec((1, _BLK, _T), lambda b, i: (b, i, 0)),
            full((_T, 2 * _T)), full((1, 2 * _T)),
            full((2 * _T, _T)), full((1, _T)),
            full((1, _T)), full((1, _T)),
            full((_T, _T)), full((1, _T)),
            full((_T, _T)), full((1, _T)),
        ],
        out_specs=pl.BlockSpec((1, _BLK, _T), lambda b, i: (b, i, 0)),
        out_shape=jax.ShapeDtypeStruct((_B, _S, _T), f32),
    )(nd, ni, topo16, topo32,
      W2.astype(bf16), b2.reshape(1, 2 * _T),
      W3.astype(bf16), b3.reshape(1, _T),
      ln_scale.reshape(1, _T), ln_bias.reshape(1, _T),
      W4.astype(bf16), b4.reshape(1, _T),
      W5.astype(bf16), b5.reshape(1, _T))
    return pf, nd, ni


# speedup vs baseline: 14.7015x; 1.2626x over previous
import functools

import jax
import jax.numpy as jnp
from jax import lax
from jax.experimental import pallas as pl
from jax.experimental.pallas import tpu as pltpu
from jax.experimental.pallas import tpu_sc as plsc

_B, _S, _D, _T, _K = 4, 2048, 1024, 512, 32
_BLK = 256
_DBLK = 512
_R = _B * _S
_NW = 32
_RPW = _R // _NW
_GRP = 16
_NG = _RPW // _GRP
_CW = 32
_NCH = _S // _CW
_BIGF = 3.0e38


def _prep_kernel(emb_ref, w1_ref, b1_ref, topo32_ref, topo16_ref):
    e = emb_ref[0]
    t = jax.lax.dot_general(
        e.astype(jnp.bfloat16), w1_ref[...], (((1,), (0,)), ((), ())),
        preferred_element_type=jnp.float32) + b1_ref[...]
    topo32_ref[0] = t
    topo16_ref[0] = t.astype(jnp.bfloat16)


def _dist_kernel(nrm_ref, nrow_ref, dist_ref):
    i = pl.program_id(1)
    sim = jax.lax.dot_general(
        nrow_ref[0], nrm_ref[0], (((1,), (1,)), ((), ())),
        preferred_element_type=jnp.float32)
    dist = 1.0 - sim
    row_ids = i * _DBLK + jax.lax.broadcasted_iota(jnp.int32, (_DBLK, _S), 0)
    col_ids = jax.lax.broadcasted_iota(jnp.int32, (_DBLK, _S), 1)
    dist_ref[0] = jnp.where(col_ids == row_ids, 1e9, dist)


def _sc_topk_body(b, dist_hbm, nd_hbm, ni_hbm, buf, cm, ndv, niv, sem):
    wid = lax.axis_index("s") * 2 + lax.axis_index("c")
    rowbase = wid * (_S // _NW)
    lane = lax.iota(jnp.int32, 16)
    rowoff = lane * _S
    zeros_i = jnp.zeros((16,), jnp.int32)
    bigv = jnp.full((16,), _BIGF, jnp.float32)

    def group_body(g, carry_g):
        base = rowbase + g * _GRP
        copies = [
            pltpu.async_copy(
                dist_hbm.at[b, base + l],
                buf.at[pl.ds(l * _S, _S)], sem)
            for l in range(_GRP)
        ]
        for cp in copies:
            cp.wait()

        def cm_body(c, carry):
            idx = rowoff + c * _CW
            accs = [bigv] * 4
            for u in range(_CW):
                v = plsc.load_gather(buf, [idx + u])
                accs[u % 4] = jnp.minimum(accs[u % 4], v)
            acc = jnp.minimum(jnp.minimum(accs[0], accs[1]),
                              jnp.minimum(accs[2], accs[3]))
            cm[pl.ds(c * _GRP, _GRP)] = acc
            return carry
        lax.fori_loop(0, _NCH, cm_body, 0)

        def t_body(t, carry_t):
            bvs, bcs = [bigv] * 4, [zeros_i] * 4
            for c in range(_NCH):
                j = c % 4
                v = cm[pl.ds(c * _GRP, _GRP)]
                lt = v < bvs[j]
                bvs[j] = jnp.where(lt, v, bvs[j])
                bcs[j] = jnp.where(lt, c + zeros_i, bcs[j])
            m, cidx = bvs[0], bcs[0]
            for j in range(1, 4):
                better = (bvs[j] < m) | ((bvs[j] == m) & (bcs[j] < cidx))
                m = jnp.where(better, bvs[j], m)
                cidx = jnp.where(better, bcs[j], cidx)
            colbase = cidx * _CW

            st = [[bigv, zeros_i, bigv] for _ in range(4)]
            for u in range(_CW):
                j = u % 4
                col = colbase + u
                v = plsc.load_gather(buf, [rowoff + col])
                b1, bcol, b2 = st[j]
                lt = v < b1
                st[j][2] = jnp.where(lt, b1, jnp.minimum(b2, v))
                st[j][0] = jnp.where(lt, v, b1)
                st[j][1] = jnp.where(lt, col, bcol)
            m1, col, nmin = st[0]
            for j in range(1, 4):
                bvv, bc2, b2v = st[j]
                better = (bvv < m1) | ((bvv == m1) & (bc2 < col))
                nmin = jnp.where(better, jnp.minimum(b2v, m1),
                                 jnp.minimum(nmin, bvv))
                m1 = jnp.where(better, bvv, m1)
                col = jnp.where(better, bc2, col)

            plsc.store_scatter(buf, [rowoff + col], bigv)
            plsc.store_scatter(cm, [cidx * _GRP + lane], nmin)
            tv = t + zeros_i
            plsc.store_scatter(ndv, [lane * _K + tv], m1)
            plsc.store_scatter(niv, [lane * _K + tv], col)
            return carry_t
        lax.fori_loop(0, _K, t_body, 0)

        pltpu.sync_copy(ndv, nd_hbm.at[pl.ds(base * _K, _GRP * _K)])
        pltpu.sync_copy(niv, ni_hbm.at[pl.ds(base * _K, _GRP * _K)])
        return carry_g
    lax.fori_loop(0, _S // _NW // _GRP, group_body, 0)


def _tail_kernel(nd_ref, ni_ref, topo16_ref, trow_ref,
                 w2_ref, b2_ref, w3_ref, b3_ref, lns_ref, lnb_ref,
                 w4_ref, b4_ref, w5_ref, b5_ref, pf_ref):
    nd = nd_ref[0]
    ni = ni_ref[0]
    col_ids = jax.lax.broadcasted_iota(jnp.int32, (_BLK, _S), 1)
    d0 = nd[:, 0:1]
    e = jnp.exp(d0 - nd)
    z = jnp.sum(e, axis=1, keepdims=True)
    w = e / z
    a = jnp.zeros((_BLK, _S), jnp.float32)
    for t in range(_K):
        a = jnp.where(col_ids == ni[:, t:t + 1], w[:, t:t + 1], a)
    abf = a.astype(jnp.bfloat16)
    wn = jax.lax.dot_general(
        abf, topo16_ref[0], (((1,), (0,)), ((), ())),
        preferred_element_type=jnp.float32)
    comb = trow_ref[0] + wn

    x = jax.lax.dot_general(
        comb.astype(jnp.bfloat16), w2_ref[...], (((1,), (0,)), ((), ())),
        preferred_element_type=jnp.float32) + b2_ref[...]
    x = jnp.maximum(x, 0.0)
    x = jax.lax.dot_general(
        x.astype(jnp.bfloat16), w3_ref[...], (((1,), (0,)), ((), ())),
        preferred_element_type=jnp.float32) + b3_ref[...]
    mu = jnp.mean(x, axis=1, keepdims=True)
    xc = x - mu
    var = jnp.mean(xc * xc, axis=1, keepdims=True)
    tf = xc / jnp.sqrt(var + 1e-6) * lns_ref[...] + lnb_ref[...]
    x = jax.lax.dot_general(
        tf.astype(jnp.bfloat16), w4_ref[...], (((1,), (0,)), ((), ())),
        preferred_element_type=jnp.float32) + b4_ref[...]
    x = jnp.maximum(x, 0.0)
    pf = jax.lax.dot_general(
        x.astype(jnp.bfloat16), w5_ref[...], (((1,), (0,)), ((), ())),
        preferred_element_type=jnp.float32) + b5_ref[...]
    pf_ref[0] = pf


def _make_sc_topk(b):
    return functools.partial(
        pl.kernel,
        out_type=[
            jax.ShapeDtypeStruct((_S * _K,), jnp.float32),
            jax.ShapeDtypeStruct((_S * _K,), jnp.int32),
        ],
        mesh=plsc.VectorSubcoreMesh(core_axis_name="c", subcore_axis_name="s"),
        compiler_params=pltpu.CompilerParams(needs_layout_passes=False),
        scratch_types=[
            pltpu.VMEM((_GRP * _S,), jnp.float32),
            pltpu.VMEM((_NCH * _GRP,), jnp.float32),
            pltpu.VMEM((_GRP * _K,), jnp.float32),
            pltpu.VMEM((_GRP * _K,), jnp.int32),
            pltpu.SemaphoreType.DMA,
        ],
    )(functools.partial(_sc_topk_body, b))


_SC_TOPK = [_make_sc_topk(b) for b in range(_B)]


def kernel(embeddings, W1, b1, W2, b2, W3, b3, ln_scale, ln_bias, W4, b4, W5, b5):
    f32, bf16 = jnp.float32, jnp.bfloat16
    norm16 = (embeddings / (jnp.linalg.norm(
        embeddings, axis=-1, keepdims=True) + 1e-08)).astype(bf16)
    topo32, topo16 = pl.pallas_call(
        _prep_kernel,
        grid=(_B,),
        in_specs=[
            pl.BlockSpec((1, _S, _D), lambda b: (b, 0, 0)),
            pl.BlockSpec((_D, _T), lambda b: (0, 0)),
            pl.BlockSpec((1, _T), lambda b: (0, 0)),
        ],
        out_specs=[
            pl.BlockSpec((1, _S, _T), lambda b: (b, 0, 0)),
            pl.BlockSpec((1, _S, _T), lambda b: (b, 0, 0)),
        ],
        out_shape=[
            jax.ShapeDtypeStruct((_B, _S, _T), f32),
            jax.ShapeDtypeStruct((_B, _S, _T), bf16),
        ],
    )(embeddings, W1.astype(bf16), b1.reshape(1, _T))

    dist = pl.pallas_call(
        _dist_kernel,
        grid=(_B, _S // _DBLK),
        in_specs=[
            pl.BlockSpec((1, _S, _D), lambda b, i: (b, 0, 0)),
            pl.BlockSpec((1, _DBLK, _D), lambda b, i: (b, i, 0)),
        ],
        out_specs=pl.BlockSpec((1, _DBLK, _S), lambda b, i: (b, i, 0)),
        out_shape=jax.ShapeDtypeStruct((_B, _S, _S), f32),
    )(norm16, norm16)

    wargs = (W2.astype(bf16), b2.reshape(1, 2 * _T),
             W3.astype(bf16), b3.reshape(1, _T),
             ln_scale.reshape(1, _T), ln_bias.reshape(1, _T),
             W4.astype(bf16), b4.reshape(1, _T),
             W5.astype(bf16), b5.reshape(1, _T))
    full = lambda shape: pl.BlockSpec(shape, lambda bb, i: tuple(0 for _ in shape))
    nds, nis, pfs = [], [], []
    for b in range(_B):
        nd_b, ni_b = _SC_TOPK[b](dist)
        nd_b = nd_b.reshape(1, _S, _K)
        ni_b = ni_b.reshape(1, _S, _K)
        pf_b = pl.pallas_call(
            _tail_kernel,
            grid=(1, _S // _BLK),
            in_specs=[
                pl.BlockSpec((1, _BLK, _K), lambda bb, i: (bb, i, 0)),
                pl.BlockSpec((1, _BLK, _K), lambda bb, i: (bb, i, 0)),
                pl.BlockSpec((1, _S, _T), lambda bb, i, b=b: (b, 0, 0)),
                pl.BlockSpec((1, _BLK, _T), lambda bb, i, b=b: (b, i, 0)),
                full((_T, 2 * _T)), full((1, 2 * _T)),
                full((2 * _T, _T)), full((1, _T)),
                full((1, _T)), full((1, _T)),
                full((_T, _T)), full((1, _T)),
                full((_T, _T)), full((1, _T)),
            ],
            out_specs=pl.BlockSpec((1, _BLK, _T), lambda bb, i: (bb, i, 0)),
            out_shape=jax.ShapeDtypeStruct((1, _S, _T), f32),
        )(nd_b, ni_b, topo16, topo32, *wargs)
        nds.append(nd_b)
        nis.append(ni_b)
        pfs.append(pf_b)
    nd = jnp.concatenate(nds)
    ni = jnp.concatenate(nis)
    pf = jnp.concatenate(pfs)
    return pf, nd, ni
